# Initial kernel scaffold; baseline (speedup 1.0000x reference)
import jax
import jax.numpy as jnp
from jax.experimental import pallas as pl


def kernel(cost_volume, hypo_coords, params):
    raise NotImplementedError("write your pallas kernel here")


# speedup vs baseline: 1.6444x; 1.6444x over previous
import functools

import jax
import jax.numpy as jnp
from jax.experimental import pallas as pl

ZGRID = 32
MARGIN = 256

_PREC = jax.lax.Precision.HIGHEST


def _dot(a, b):
    return jax.lax.dot_general(a, b, (((1,), (0,)), ((), ())),
                               preferred_element_type=jnp.float32,
                               precision=_PREC)


def _to_T(x):
    Z, H, W, C = x.shape
    xp = jnp.pad(x, ((1, 1), (1, 1), (1, 1), (0, 0)))
    xt = xp.transpose(0, 3, 1, 2).reshape(Z + 2, C, (H + 2) * (W + 2))
    return jnp.pad(xt, ((0, 0), (0, 0), (MARGIN, MARGIN)))


def _from_T(xt, Z, H, W):
    C = xt.shape[1]
    xs = xt[1:Z + 1, :, MARGIN:MARGIN + (H + 2) * (W + 2)]
    xs = xs.reshape(Z, C, H + 2, W + 2)[:, :, 1:-1, 1:-1]
    return xs.transpose(0, 2, 3, 1)


def _to_dense_T(xt, Z, H, W):
    C = xt.shape[1]
    xs = xt[1:Z + 1, :, MARGIN:MARGIN + (H + 2) * (W + 2)]
    xs = xs.reshape(Z, C, H + 2, W + 2)[:, :, 1:-1, 1:-1]
    return xs.reshape(Z, C, H * W)


def _dense_T_to_T(xd, Z, H, W):
    C = xd.shape[1]
    x = xd.reshape(Z, C, H, W)
    x = jnp.pad(x, ((1, 1), (0, 0), (1, 1), (1, 1)))
    x = x.reshape(Z + 2, C, (H + 2) * (W + 2))
    return jnp.pad(x, ((0, 0), (0, 0), (MARGIN, MARGIN)))


def _dense_T_to_dense(xd, Z, H, W):
    C = xd.shape[1]
    return xd.reshape(Z, C, H, W).transpose(0, 2, 3, 1)


def _to_taps_T(x):
    Z2, H2, W2, C = x.shape
    Z, H, W = Z2 // 2, H2 // 2, W2 // 2
    xr = x.reshape(Z, 2, H, 2, W, 2, C).transpose(1, 3, 5, 0, 6, 2, 4)
    return xr.reshape(8, Z, C, H * W)


def _untaps_T(y8, Z, H, W):
    C = y8.shape[2]
    yr = y8.reshape(2, 2, 2, Z, C, H, W).transpose(3, 0, 5, 1, 6, 2, 4)
    return yr.reshape(2 * Z, 2 * H, 2 * W, C)


def _scatter_body(zp_ref, fp_ref, vol_ref, m_ref):
    z = (pl.program_id(0) - 1).astype(jnp.float32)
    D = zp_ref.shape[0]
    acc = jnp.zeros(vol_ref.shape[1:], jnp.float32)
    mm = jnp.zeros(m_ref.shape[1:], jnp.float32)
    for d in range(D):
        match = zp_ref[d] == z
        acc = jnp.where(match, fp_ref[d], acc)
        mm = jnp.maximum(mm, match.astype(jnp.float32))
    vol_ref[0] = acc
    m_ref[0] = mm


def _scatter(zp, fp):
    D, C, Sp = zp.shape
    G = ZGRID + 2
    return pl.pallas_call(
        _scatter_body,
        grid=(G,),
        in_specs=[
            pl.BlockSpec((D, C, Sp), lambda j: (0, 0, 0)),
            pl.BlockSpec((D, C, Sp), lambda j: (0, 0, 0)),
        ],
        out_specs=[
            pl.BlockSpec((1, C, Sp), lambda j: (j, 0, 0)),
            pl.BlockSpec((1, C, Sp), lambda j: (j, 0, 0)),
        ],
        out_shape=[
            jax.ShapeDtypeStruct((G, C, Sp), jnp.float32),
            jax.ShapeDtypeStruct((G, C, Sp), jnp.float32),
        ],
    )(zp, fp)


def _fconv_body(*refs, ioffs, kd, Z, S, relu, bias, has_skip):
    if has_skip:
        x_ref, w_ref, b_ref, m_ref, s_ref, o_ref = refs
    else:
        x_ref, w_ref, b_ref, m_ref, o_ref = refs
    j = pl.program_id(0)
    rows = []
    for dz in range(kd):
        zi = jnp.clip(j + dz - kd // 2, 0, Z + 1)
        xz = x_ref[pl.ds(zi, 1), :, :][0]
        for o in ioffs:
            rows.append(xz[:, MARGIN + o:MARGIN + o + S])
    xcat = rows[0] if len(rows) == 1 else jnp.concatenate(rows, axis=0)
    y = _dot(w_ref[...], xcat)
    if bias:
        y = y + b_ref[...]
    if relu:
        y = jnp.maximum(y, 0.0)
    y = y * m_ref[0, :, MARGIN:MARGIN + S]
    if has_skip:
        y = y + s_ref[0, :, MARGIN:MARGIN + S]
    o_ref[0, :, :MARGIN] = jnp.zeros_like(o_ref[0, :, :MARGIN])
    o_ref[0, :, MARGIN:MARGIN + S] = y
    o_ref[0, :, MARGIN + S:] = jnp.zeros_like(o_ref[0, :, MARGIN + S:])


def _fconv(xt, w, b, mt, Z, H, W, relu=True, skip=None):
    kd, kh, kw, Ci, Co = w.shape
    Wp = W + 2
    S = (H + 2) * Wp
    Sp = xt.shape[2]
    ioffs = [(dy - kh // 2) * Wp + (dx - kw // 2)
             for dy in range(kh) for dx in range(kw)]
    wt = w.reshape(kd * kh * kw * Ci, Co).T
    has_bias = b is not None
    bb = b.reshape(Co, 1) if has_bias else jnp.zeros((Co, 1), jnp.float32)
    body = functools.partial(_fconv_body, ioffs=ioffs, kd=kd, Z=Z, S=S,
                             relu=relu, bias=has_bias,
                             has_skip=skip is not None)
    G = Z + 2
    in_specs = [
        pl.BlockSpec(xt.shape, lambda j: (0, 0, 0)),
        pl.BlockSpec(wt.shape, lambda j: (0, 0)),
        pl.BlockSpec(bb.shape, lambda j: (0, 0)),
        pl.BlockSpec((1, Co, Sp), lambda j: (j, 0, 0)),
    ]
    args = [xt, wt, bb, mt]
    if skip is not None:
        in_specs.append(pl.BlockSpec((1, Co, Sp), lambda j: (j, 0, 0)))
        args.append(skip)
    return pl.pallas_call(
        body,
        grid=(G,),
        in_specs=in_specs,
        out_specs=pl.BlockSpec((1, Co, Sp), lambda j: (j, 0, 0)),
        out_shape=jax.ShapeDtypeStruct((G, Co, Sp), jnp.float32),
    )(*args)


def _down_body(x8_ref, w_ref, b_ref, m_ref, o_ref):
    xs = jnp.concatenate([x8_ref[t, 0] for t in range(8)], axis=0)
    y = jnp.maximum(_dot(w_ref[...], xs) + b_ref[...], 0.0)
    o_ref[0] = y * m_ref[0]


def _down(x_dense, w, b, m_dT):
    x8 = _to_taps_T(x_dense)
    _, Zo, Ci, N = x8.shape
    Co = w.shape[-1]
    wt = w.reshape(8 * Ci, Co).T
    bb = b.reshape(Co, 1)
    return pl.pallas_call(
        _down_body,
        grid=(Zo,),
        in_specs=[
            pl.BlockSpec((8, 1, Ci, N), lambda z: (0, z, 0, 0)),
            pl.BlockSpec(wt.shape, lambda z: (0, 0)),
            pl.BlockSpec(bb.shape, lambda z: (0, 0)),
            pl.BlockSpec((1, Co, N), lambda z: (z, 0, 0)),
        ],
        out_specs=pl.BlockSpec((1, Co, N), lambda z: (z, 0, 0)),
        out_shape=jax.ShapeDtypeStruct((Zo, Co, N), jnp.float32),
    )(x8, wt, bb, m_dT)


def _up_body(x_ref, w_ref, b_ref, m8_ref, o_ref):
    xs = x_ref[0]
    for t in range(8):
        y = jnp.maximum(_dot(w_ref[t], xs) + b_ref[...], 0.0)
        o_ref[t, 0] = y * m8_ref[t, 0]


def _up(x_dT, w, b, m8T):
    Zi, Ci, N = x_dT.shape
    Co = w.shape[-1]
    wt = w[::-1, ::-1, ::-1].reshape(8, Ci, Co).transpose(0, 2, 1)
    bb = b.reshape(Co, 1)
    return pl.pallas_call(
        _up_body,
        grid=(Zi,),
        in_specs=[
            pl.BlockSpec((1, Ci, N), lambda z: (z, 0, 0)),
            pl.BlockSpec(wt.shape, lambda z: (0, 0, 0)),
            pl.BlockSpec(bb.shape, lambda z: (0, 0)),
            pl.BlockSpec((8, 1, Co, N), lambda z: (0, z, 0, 0)),
        ],
        out_specs=pl.BlockSpec((8, 1, Co, N), lambda z: (0, z, 0, 0)),
        out_shape=jax.ShapeDtypeStruct((8, Zi, Co, N), jnp.float32),
    )(x_dT, wt, bb, m8T)


def _pool_body(m8_ref, o_ref, *, widen):
    acc = m8_ref[0, 0]
    for t in range(1, 8):
        acc = jnp.maximum(acc, m8_ref[t, 0])
    if widen > 1:
        acc = jnp.concatenate([acc] * widen, axis=0)
    o_ref[0] = acc


def _pool(m_dense, c_out):
    m8 = _to_taps_T(m_dense)
    _, Zo, Ci, N = m8.shape
    body = functools.partial(_pool_body, widen=c_out // Ci)
    return pl.pallas_call(
        body,
        grid=(Zo,),
        in_specs=[pl.BlockSpec((8, 1, Ci, N), lambda z: (0, z, 0, 0))],
        out_specs=pl.BlockSpec((1, c_out, N), lambda z: (z, 0, 0)),
        out_shape=jax.ShapeDtypeStruct((Zo, c_out, N), jnp.float32),
    )(m8)


def _gather_body(zp_ref, x_ref, wf_ref, o_ref):
    D, C, Sp = zp_ref.shape

    def zstep(z, acc, zi):
        xs = x_ref[pl.ds(z + 1, 1), :, :][0]
        return jnp.where(zi == z.astype(jnp.float32), xs, acc)

    for d in range(D):
        zi = zp_ref[d]
        acc = jax.lax.fori_loop(
            0, ZGRID, lambda z, a: zstep(z, a, zi),
            jnp.zeros((C, Sp), jnp.float32))
        o_ref[d] = jnp.sum(acc * wf_ref[...], axis=0)


def _gather(zp, xt, wf):
    D, C, Sp = zp.shape
    G = xt.shape[0]
    return pl.pallas_call(
        _gather_body,
        in_specs=[
            pl.BlockSpec((D, C, Sp), lambda: (0, 0, 0)),
            pl.BlockSpec((G, C, Sp), lambda: (0, 0, 0)),
            pl.BlockSpec((C, 1), lambda: (0, 0)),
        ],
        out_specs=pl.BlockSpec((D, Sp), lambda: (0, 0)),
        out_shape=jax.ShapeDtypeStruct((D, Sp), jnp.float32),
    )(zp, xt, wf.reshape(C, 1))


def kernel(cost_volume, hypo_coords, params):
    p = params
    Bc, C, D, H, W = cost_volume.shape
    Hp, Wp = H + 2, W + 2
    S = Hp * Wp

    featsT = cost_volume[0].transpose(1, 0, 2, 3)
    fp = jnp.pad(featsT, ((0, 0), (0, 0), (1, 1), (1, 1)))
    fp = jnp.pad(fp.reshape(D, C, S), ((0, 0), (0, 0), (MARGIN, MARGIN)))
    zidx = jnp.clip(hypo_coords[0, 0].astype(jnp.int32), 0, ZGRID - 1)
    zidxT = jnp.broadcast_to(zidx[:, None], (D, C, H, W)).astype(jnp.float32)
    zp = jnp.pad(zidxT, ((0, 0), (0, 0), (1, 1), (1, 1)),
                 constant_values=-128.0)
    zp = jnp.pad(zp.reshape(D, C, S), ((0, 0), (0, 0), (MARGIN, MARGIN)),
                 constant_values=-128.0)

    vol, m0 = _scatter(zp, fp)
    m0d = _from_T(m0, ZGRID, H, W)

    Z1, H1, W1 = ZGRID // 2, H // 2, W // 2
    Z2, H2, W2 = Z1 // 2, H1 // 2, W1 // 2
    Z3, H3, W3 = Z2 // 2, H2 // 2, W2 // 2

    m1dT = _pool(m0d, 16)
    m1d = _dense_T_to_dense(m1dT, Z1, H1, W1)
    m2dT = _pool(m1d, 32)
    m2d = _dense_T_to_dense(m2dT, Z2, H2, W2)
    m3dT = _pool(m2d, 64)
    m1 = _dense_T_to_T(m1dT, Z1, H1, W1)
    m2 = _dense_T_to_T(m2dT, Z2, H2, W2)
    m3 = _dense_T_to_T(m3dT, Z3, H3, W3)

    def blk(xt, name, mt, Z_, H_, W_, skip=None):
        return _fconv(xt, p[name], p[name + '_b'], mt, Z_, H_, W_, skip=skip)

    x = blk(vol, 'in0', m0, ZGRID, H, W)
    x = blk(x, 'in1', m0, ZGRID, H, W)
    conv0 = blk(x, 'in2', m0, ZGRID, H, W)
    x = _down(_from_T(conv0, ZGRID, H, W), p['d1'], p['d1_b'], m1dT)
    x = _dense_T_to_T(x, Z1, H1, W1)
    x = blk(x, 'c1a', m1, Z1, H1, W1)
    conv1up = blk(x, 'c1b', m1, Z1, H1, W1)
    x = _down(_from_T(conv1up, Z1, H1, W1), p['d2'], p['d2_b'], m2dT)
    x = _dense_T_to_T(x, Z2, H2, W2)
    x = blk(x, 'c2a', m2, Z2, H2, W2)
    conv2up = blk(x, 'c2b', m2, Z2, H2, W2)
    x = _down(_from_T(conv2up, Z2, H2, W2), p['d3'], p['d3_b'], m3dT)
    x = _dense_T_to_T(x, Z3, H3, W3)
    x = blk(x, 'c3a', m3, Z3, H3, W3)
    conv3up = blk(x, 'c3b', m3, Z3, H3, W3)

    y8 = _up(_to_dense_T(conv3up, Z3, H3, W3), p['u3'], p['u3_b'],
             _to_taps_T(m2d))
    y = _to_T(_untaps_T(y8, Z3, H3, W3))
    y = blk(y, 'u3a', m2, Z2, H2, W2)
    conv3down = blk(y, 'u3b', m2, Z2, H2, W2, skip=conv2up)
    y8 = _up(_to_dense_T(conv3down, Z2, H2, W2), p['u2'], p['u2_b'],
             _to_taps_T(m1d))
    y = _to_T(_untaps_T(y8, Z2, H2, W2))
    y = blk(y, 'u2a', m1, Z1, H1, W1)
    conv2down = blk(y, 'u2b', m1, Z1, H1, W1, skip=conv1up)
    y8 = _up(_to_dense_T(conv2down, Z1, H1, W1), p['u1'], p['u1_b'],
             _to_taps_T(m0d))
    y = _to_T(_untaps_T(y8, Z1, H1, W1))
    y = blk(y, 'u1a', m0, ZGRID, H, W)
    conv1down = blk(y, 'u1b', m0, ZGRID, H, W, skip=conv0)

    x = blk(conv1down, 'p0', m0, ZGRID, H, W)
    x = blk(x, 'p1', m0, ZGRID, H, W)
    x = _fconv(x, p['pz'], None, m0, ZGRID, H, W, relu=False)
    x = _fconv(x, p['pw'], None, m0, ZGRID, H, W, relu=False)
    x = _fconv(x, p['ph'], None, m0, ZGRID, H, W, relu=False)

    est = _gather(zp, x, p['pf'].reshape(C))
    est = est[:, MARGIN:MARGIN + S].reshape(D, Hp, Wp)[:, 1:-1, 1:-1]
    return est.reshape(Bc, 1, D, H, W)


# speedup vs baseline: 2.3625x; 1.4367x over previous
import functools

import jax
import jax.numpy as jnp
from jax.experimental import pallas as pl

ZGRID = 32
MARGIN = 256

_PREC = jax.lax.Precision.DEFAULT


def _dot(a, b):
    return jax.lax.dot_general(a, b, (((1,), (0,)), ((), ())),
                               preferred_element_type=jnp.float32,
                               precision=_PREC)


def _to_T(x):
    Z, H, W, C = x.shape
    xp = jnp.pad(x, ((1, 1), (1, 1), (1, 1), (0, 0)))
    xt = xp.transpose(0, 3, 1, 2).reshape(Z + 2, C, (H + 2) * (W + 2))
    return jnp.pad(xt, ((0, 0), (0, 0), (MARGIN, MARGIN)))


def _from_T(xt, Z, H, W):
    C = xt.shape[1]
    xs = xt[1:Z + 1, :, MARGIN:MARGIN + (H + 2) * (W + 2)]
    xs = xs.reshape(Z, C, H + 2, W + 2)[:, :, 1:-1, 1:-1]
    return xs.transpose(0, 2, 3, 1)


def _to_dense_T(xt, Z, H, W):
    C = xt.shape[1]
    xs = xt[1:Z + 1, :, MARGIN:MARGIN + (H + 2) * (W + 2)]
    xs = xs.reshape(Z, C, H + 2, W + 2)[:, :, 1:-1, 1:-1]
    return xs.reshape(Z, C, H * W)


def _dense_T_to_T(xd, Z, H, W):
    C = xd.shape[1]
    x = xd.reshape(Z, C, H, W)
    x = jnp.pad(x, ((1, 1), (0, 0), (1, 1), (1, 1)))
    x = x.reshape(Z + 2, C, (H + 2) * (W + 2))
    return jnp.pad(x, ((0, 0), (0, 0), (MARGIN, MARGIN)))


def _dense_T_to_dense(xd, Z, H, W):
    C = xd.shape[1]
    return xd.reshape(Z, C, H, W).transpose(0, 2, 3, 1)


def _to_taps_T(x):
    Z2, H2, W2, C = x.shape
    Z, H, W = Z2 // 2, H2 // 2, W2 // 2
    xr = x.reshape(Z, 2, H, 2, W, 2, C).transpose(1, 3, 5, 0, 6, 2, 4)
    return xr.reshape(8, Z, C, H * W)


def _untaps_T(y8, Z, H, W):
    C = y8.shape[2]
    yr = y8.reshape(2, 2, 2, Z, C, H, W).transpose(3, 0, 5, 1, 6, 2, 4)
    return yr.reshape(2 * Z, 2 * H, 2 * W, C)


def _scatter_body(zp_ref, fp_ref, vol_ref, m_ref):
    z = (pl.program_id(0) - 1).astype(jnp.float32)
    D = zp_ref.shape[0]
    acc = jnp.zeros(vol_ref.shape[1:], jnp.float32)
    mm = jnp.zeros(m_ref.shape[1:], jnp.float32)
    for d in range(D):
        match = zp_ref[d] == z
        acc = jnp.where(match, fp_ref[d], acc)
        mm = jnp.maximum(mm, match.astype(jnp.float32))
    vol_ref[0] = acc
    m_ref[0] = mm


def _scatter(zp, fp):
    D, C, Sp = zp.shape
    G = ZGRID + 2
    return pl.pallas_call(
        _scatter_body,
        grid=(G,),
        in_specs=[
            pl.BlockSpec((D, C, Sp), lambda j: (0, 0, 0)),
            pl.BlockSpec((D, C, Sp), lambda j: (0, 0, 0)),
        ],
        out_specs=[
            pl.BlockSpec((1, C, Sp), lambda j: (j, 0, 0)),
            pl.BlockSpec((1, C, Sp), lambda j: (j, 0, 0)),
        ],
        out_shape=[
            jax.ShapeDtypeStruct((G, C, Sp), jnp.float32),
            jax.ShapeDtypeStruct((G, C, Sp), jnp.float32),
        ],
    )(zp, fp)


def _fconv_body(*refs, ioffs, kd, Z, S, relu, bias, has_skip):
    if has_skip:
        x_ref, w_ref, b_ref, m_ref, s_ref, o_ref = refs
    else:
        x_ref, w_ref, b_ref, m_ref, o_ref = refs
    j = pl.program_id(0)
    rows = []
    for dz in range(kd):
        zi = jnp.clip(j + dz - kd // 2, 0, Z + 1)
        xz = x_ref[pl.ds(zi, 1), :, :][0]
        for o in ioffs:
            rows.append(xz[:, MARGIN + o:MARGIN + o + S])
    xcat = rows[0] if len(rows) == 1 else jnp.concatenate(rows, axis=0)
    y = _dot(w_ref[...], xcat)
    if bias:
        y = y + b_ref[...]
    if relu:
        y = jnp.maximum(y, 0.0)
    y = y * m_ref[0, :, MARGIN:MARGIN + S]
    if has_skip:
        y = y + s_ref[0, :, MARGIN:MARGIN + S]
    o_ref[0, :, :MARGIN] = jnp.zeros_like(o_ref[0, :, :MARGIN])
    o_ref[0, :, MARGIN:MARGIN + S] = y
    o_ref[0, :, MARGIN + S:] = jnp.zeros_like(o_ref[0, :, MARGIN + S:])


def _fconv(xt, w, b, mt, Z, H, W, relu=True, skip=None):
    kd, kh, kw, Ci, Co = w.shape
    Wp = W + 2
    S = (H + 2) * Wp
    Sp = xt.shape[2]
    ioffs = [(dy - kh // 2) * Wp + (dx - kw // 2)
             for dy in range(kh) for dx in range(kw)]
    wt = w.reshape(kd * kh * kw * Ci, Co).T
    has_bias = b is not None
    bb = b.reshape(Co, 1) if has_bias else jnp.zeros((Co, 1), jnp.float32)
    body = functools.partial(_fconv_body, ioffs=ioffs, kd=kd, Z=Z, S=S,
                             relu=relu, bias=has_bias,
                             has_skip=skip is not None)
    G = Z + 2
    in_specs = [
        pl.BlockSpec(xt.shape, lambda j: (0, 0, 0)),
        pl.BlockSpec(wt.shape, lambda j: (0, 0)),
        pl.BlockSpec(bb.shape, lambda j: (0, 0)),
        pl.BlockSpec((1, Co, Sp), lambda j: (j, 0, 0)),
    ]
    args = [xt, wt, bb, mt]
    if skip is not None:
        in_specs.append(pl.BlockSpec((1, Co, Sp), lambda j: (j, 0, 0)))
        args.append(skip)
    return pl.pallas_call(
        body,
        grid=(G,),
        in_specs=in_specs,
        out_specs=pl.BlockSpec((1, Co, Sp), lambda j: (j, 0, 0)),
        out_shape=jax.ShapeDtypeStruct((G, Co, Sp), jnp.float32),
    )(*args)


def _down_body(x8_ref, w_ref, b_ref, m_ref, o_ref):
    xs = jnp.concatenate([x8_ref[t, 0] for t in range(8)], axis=0)
    y = jnp.maximum(_dot(w_ref[...], xs) + b_ref[...], 0.0)
    o_ref[0] = y * m_ref[0]


def _down(x_dense, w, b, m_dT):
    x8 = _to_taps_T(x_dense)
    _, Zo, Ci, N = x8.shape
    Co = w.shape[-1]
    wt = w.reshape(8 * Ci, Co).T
    bb = b.reshape(Co, 1)
    return pl.pallas_call(
        _down_body,
        grid=(Zo,),
        in_specs=[
            pl.BlockSpec((8, 1, Ci, N), lambda z: (0, z, 0, 0)),
            pl.BlockSpec(wt.shape, lambda z: (0, 0)),
            pl.BlockSpec(bb.shape, lambda z: (0, 0)),
            pl.BlockSpec((1, Co, N), lambda z: (z, 0, 0)),
        ],
        out_specs=pl.BlockSpec((1, Co, N), lambda z: (z, 0, 0)),
        out_shape=jax.ShapeDtypeStruct((Zo, Co, N), jnp.float32),
    )(x8, wt, bb, m_dT)


def _up_body(x_ref, w_ref, b_ref, m8_ref, o_ref):
    xs = x_ref[0]
    for t in range(8):
        y = jnp.maximum(_dot(w_ref[t], xs) + b_ref[...], 0.0)
        o_ref[t, 0] = y * m8_ref[t, 0]


def _up(x_dT, w, b, m8T):
    Zi, Ci, N = x_dT.shape
    Co = w.shape[-1]
    wt = w[::-1, ::-1, ::-1].reshape(8, Ci, Co).transpose(0, 2, 1)
    bb = b.reshape(Co, 1)
    return pl.pallas_call(
        _up_body,
        grid=(Zi,),
        in_specs=[
            pl.BlockSpec((1, Ci, N), lambda z: (z, 0, 0)),
            pl.BlockSpec(wt.shape, lambda z: (0, 0, 0)),
            pl.BlockSpec(bb.shape, lambda z: (0, 0)),
            pl.BlockSpec((8, 1, Co, N), lambda z: (0, z, 0, 0)),
        ],
        out_specs=pl.BlockSpec((8, 1, Co, N), lambda z: (0, z, 0, 0)),
        out_shape=jax.ShapeDtypeStruct((8, Zi, Co, N), jnp.float32),
    )(x_dT, wt, bb, m8T)


def _pool_body(m8_ref, o_ref, *, widen):
    acc = m8_ref[0, 0]
    for t in range(1, 8):
        acc = jnp.maximum(acc, m8_ref[t, 0])
    if widen > 1:
        acc = jnp.concatenate([acc] * widen, axis=0)
    o_ref[0] = acc


def _pool(m_dense, c_out):
    m8 = _to_taps_T(m_dense)
    _, Zo, Ci, N = m8.shape
    body = functools.partial(_pool_body, widen=c_out // Ci)
    return pl.pallas_call(
        body,
        grid=(Zo,),
        in_specs=[pl.BlockSpec((8, 1, Ci, N), lambda z: (0, z, 0, 0))],
        out_specs=pl.BlockSpec((1, c_out, N), lambda z: (z, 0, 0)),
        out_shape=jax.ShapeDtypeStruct((Zo, c_out, N), jnp.float32),
    )(m8)


def _gather_body(zp_ref, x_ref, wf_ref, o_ref):
    D, C, Sp = zp_ref.shape

    def zstep(z, acc, zi):
        xs = x_ref[pl.ds(z + 1, 1), :, :][0]
        return jnp.where(zi == z.astype(jnp.float32), xs, acc)

    for d in range(D):
        zi = zp_ref[d]
        acc = jax.lax.fori_loop(
            0, ZGRID, lambda z, a: zstep(z, a, zi),
            jnp.zeros((C, Sp), jnp.float32))
        o_ref[d] = jnp.sum(acc * wf_ref[...], axis=0)


def _gather(zp, xt, wf):
    D, C, Sp = zp.shape
    G = xt.shape[0]
    return pl.pallas_call(
        _gather_body,
        in_specs=[
            pl.BlockSpec((D, C, Sp), lambda: (0, 0, 0)),
            pl.BlockSpec((G, C, Sp), lambda: (0, 0, 0)),
            pl.BlockSpec((C, 1), lambda: (0, 0)),
        ],
        out_specs=pl.BlockSpec((D, Sp), lambda: (0, 0)),
        out_shape=jax.ShapeDtypeStruct((D, Sp), jnp.float32),
    )(zp, xt, wf.reshape(C, 1))


def kernel(cost_volume, hypo_coords, params):
    p = params
    Bc, C, D, H, W = cost_volume.shape
    Hp, Wp = H + 2, W + 2
    S = Hp * Wp

    featsT = cost_volume[0].transpose(1, 0, 2, 3)
    fp = jnp.pad(featsT, ((0, 0), (0, 0), (1, 1), (1, 1)))
    fp = jnp.pad(fp.reshape(D, C, S), ((0, 0), (0, 0), (MARGIN, MARGIN)))
    zidx = jnp.clip(hypo_coords[0, 0].astype(jnp.int32), 0, ZGRID - 1)
    zidxT = jnp.broadcast_to(zidx[:, None], (D, C, H, W)).astype(jnp.float32)
    zp = jnp.pad(zidxT, ((0, 0), (0, 0), (1, 1), (1, 1)),
                 constant_values=-128.0)
    zp = jnp.pad(zp.reshape(D, C, S), ((0, 0), (0, 0), (MARGIN, MARGIN)),
                 constant_values=-128.0)

    vol, m0 = _scatter(zp, fp)
    m0d = _from_T(m0, ZGRID, H, W)

    Z1, H1, W1 = ZGRID // 2, H // 2, W // 2
    Z2, H2, W2 = Z1 // 2, H1 // 2, W1 // 2
    Z3, H3, W3 = Z2 // 2, H2 // 2, W2 // 2

    m1dT = _pool(m0d, 16)
    m1d = _dense_T_to_dense(m1dT, Z1, H1, W1)
    m2dT = _pool(m1d, 32)
    m2d = _dense_T_to_dense(m2dT, Z2, H2, W2)
    m3dT = _pool(m2d, 64)
    m1 = _dense_T_to_T(m1dT, Z1, H1, W1)
    m2 = _dense_T_to_T(m2dT, Z2, H2, W2)
    m3 = _dense_T_to_T(m3dT, Z3, H3, W3)

    def blk(xt, name, mt, Z_, H_, W_, skip=None):
        return _fconv(xt, p[name], p[name + '_b'], mt, Z_, H_, W_, skip=skip)

    x = blk(vol, 'in0', m0, ZGRID, H, W)
    x = blk(x, 'in1', m0, ZGRID, H, W)
    conv0 = blk(x, 'in2', m0, ZGRID, H, W)
    x = _down(_from_T(conv0, ZGRID, H, W), p['d1'], p['d1_b'], m1dT)
    x = _dense_T_to_T(x, Z1, H1, W1)
    x = blk(x, 'c1a', m1, Z1, H1, W1)
    conv1up = blk(x, 'c1b', m1, Z1, H1, W1)
    x = _down(_from_T(conv1up, Z1, H1, W1), p['d2'], p['d2_b'], m2dT)
    x = _dense_T_to_T(x, Z2, H2, W2)
    x = blk(x, 'c2a', m2, Z2, H2, W2)
    conv2up = blk(x, 'c2b', m2, Z2, H2, W2)
    x = _down(_from_T(conv2up, Z2, H2, W2), p['d3'], p['d3_b'], m3dT)
    x = _dense_T_to_T(x, Z3, H3, W3)
    x = blk(x, 'c3a', m3, Z3, H3, W3)
    conv3up = blk(x, 'c3b', m3, Z3, H3, W3)

    y8 = _up(_to_dense_T(conv3up, Z3, H3, W3), p['u3'], p['u3_b'],
             _to_taps_T(m2d))
    y = _to_T(_untaps_T(y8, Z3, H3, W3))
    y = blk(y, 'u3a', m2, Z2, H2, W2)
    conv3down = blk(y, 'u3b', m2, Z2, H2, W2, skip=conv2up)
    y8 = _up(_to_dense_T(conv3down, Z2, H2, W2), p['u2'], p['u2_b'],
             _to_taps_T(m1d))
    y = _to_T(_untaps_T(y8, Z2, H2, W2))
    y = blk(y, 'u2a', m1, Z1, H1, W1)
    conv2down = blk(y, 'u2b', m1, Z1, H1, W1, skip=conv1up)
    y8 = _up(_to_dense_T(conv2down, Z1, H1, W1), p['u1'], p['u1_b'],
             _to_taps_T(m0d))
    y = _to_T(_untaps_T(y8, Z1, H1, W1))
    y = blk(y, 'u1a', m0, ZGRID, H, W)
    conv1down = blk(y, 'u1b', m0, ZGRID, H, W, skip=conv0)

    x = blk(conv1down, 'p0', m0, ZGRID, H, W)
    x = blk(x, 'p1', m0, ZGRID, H, W)
    x = _fconv(x, p['pz'], None, m0, ZGRID, H, W, relu=False)
    x = _fconv(x, p['pw'], None, m0, ZGRID, H, W, relu=False)
    x = _fconv(x, p['ph'], None, m0, ZGRID, H, W, relu=False)

    est = _gather(zp, x, p['pf'].reshape(C))
    est = est[:, MARGIN:MARGIN + S].reshape(D, Hp, Wp)[:, 1:-1, 1:-1]
    return est.reshape(Bc, 1, D, H, W)
